# baseline (speedup 1.0000x reference)
import jax
import jax.numpy as jnp
from jax import lax
from jax.experimental import pallas as pl
from jax.experimental.pallas import tpu as pltpu
from jax.experimental.pallas import tpu_sc as plsc

BATCH = 16384
EMBED_DIM = 32
NUM_CORES = 2
NUM_SUBCORES = 16
NUM_WORKERS = NUM_CORES * NUM_SUBCORES
B_PER_W = BATCH // NUM_WORKERS
LANES = 16
N_BLOCKS = B_PER_W // LANES
HALF = EMBED_DIM // LANES


def _scalar_at(vec, lane_iota, t):
    return jnp.sum(jnp.where(lane_iota == t, vec, 0))


def _fire(tab_hbm, ring_v, sems, pages, lane_iota, t):
    page_t = pl.multiple_of(_scalar_at(pages, lane_iota, t), 128)
    for blk in range(EMBED_DIM // 8):
        pltpu.async_copy(
            tab_hbm.at[pl.ds(blk * 8, 8), pl.ds(page_t, 128)],
            ring_v.at[pl.ds(t * EMBED_DIM + blk * 8, 8), :], sems[t])


def _wait(tab_hbm, ring_v, sems, t):
    pltpu.make_async_copy(
        tab_hbm.at[:, pl.ds(0, 128)],
        ring_v.at[pl.ds(t * EMBED_DIM, EMBED_DIM), :], sems[t]).wait()


def _extract_pass(tab_hbm, idx_v, ring_v, vec_v, sems):
    lane_iota = lax.iota(jnp.int32, LANES)

    def pages_of(b):
        return (idx_v[pl.ds(b * LANES, LANES)] >> 7) << 7

    pages0 = pages_of(0)
    for t in range(LANES):
        _fire(tab_hbm, ring_v, sems, pages0, lane_iota, t)

    def blk_body(b, carry):
        lanes = idx_v[pl.ds(b * LANES, LANES)] & 127
        b_next = jnp.minimum(b + 1, N_BLOCKS - 1)
        pages_next = pages_of(b_next)
        for t in range(LANES):
            _wait(tab_hbm, ring_v, sems, t)
            lane_t = _scalar_at(lanes, lane_iota, t)
            cols = jnp.full((LANES,), 0, jnp.int32) + lane_t
            base = (b * LANES + t) * EMBED_DIM
            for h in range(HALF):
                rows = t * EMBED_DIM + h * LANES + lane_iota
                vals = plsc.load_gather(ring_v, [rows, cols])
                vec_v[pl.ds(base + h * LANES, LANES)] = vals

            @pl.when(b < N_BLOCKS - 1)
            def _():
                _fire(tab_hbm, ring_v, sems, pages_next, lane_iota, t)
        return carry

    lax.fori_loop(0, N_BLOCKS, blk_body, 0)


def _sc_body(u_hbm, i_hbm, ut_hbm, it_hbm, out_hbm,
             idx_u_v, idx_i_v, ring_v, uv_v, iv_v, out_v,
             sem_o, *sems):
    wid = lax.axis_index("s") * NUM_CORES + lax.axis_index("c")
    base = wid * B_PER_W

    pltpu.sync_copy(u_hbm.at[pl.ds(base, B_PER_W)], idx_u_v)
    pltpu.sync_copy(i_hbm.at[pl.ds(base, B_PER_W)], idx_i_v)

    _extract_pass(ut_hbm, idx_u_v, ring_v, uv_v, sems)
    _extract_pass(it_hbm, idx_i_v, ring_v, iv_v, sems)

    def dot_body(b, carry):
        rows = (b * LANES + lax.iota(jnp.int32, LANES)) * EMBED_DIM
        acc = jnp.zeros((LANES,), jnp.float32)
        for d in range(EMBED_DIM):
            uvals = plsc.load_gather(uv_v, [rows + d])
            ivals = plsc.load_gather(iv_v, [rows + d])
            acc = acc + uvals * ivals
        out_v[pl.ds(b * LANES, LANES)] = 1.0 / (1.0 + jnp.exp(-acc))
        return carry

    lax.fori_loop(0, N_BLOCKS, dot_body, 0)

    pltpu.async_copy(out_v, out_hbm.at[pl.ds(base, B_PER_W)], sem_o).wait()


@jax.jit
def _mf_sc(u, i, ut_t, it_t):
    mesh = plsc.VectorSubcoreMesh(core_axis_name="c", subcore_axis_name="s")
    return pl.kernel(
        _sc_body,
        out_type=jax.ShapeDtypeStruct((BATCH,), jnp.float32),
        mesh=mesh,
        scratch_types=[
            pltpu.VMEM((B_PER_W,), jnp.int32),
            pltpu.VMEM((B_PER_W,), jnp.int32),
            pltpu.VMEM((LANES * EMBED_DIM, 128), jnp.float32),
            pltpu.VMEM((B_PER_W * EMBED_DIM,), jnp.float32),
            pltpu.VMEM((B_PER_W * EMBED_DIM,), jnp.float32),
            pltpu.VMEM((B_PER_W,), jnp.float32),
            pltpu.SemaphoreType.DMA,
        ] + [pltpu.SemaphoreType.DMA] * LANES,
        compiler_params=pltpu.CompilerParams(
            needs_layout_passes=False,
            use_tc_tiling_on_sc=True,
            disable_bounds_checks=True,
        ),
    )(u, i, ut_t, it_t)


def kernel(u, i, user_table, item_table):
    return _mf_sc(u, i, user_table.T, item_table.T)
